# baseline (speedup 1.0000x reference)
import functools

import jax
import jax.numpy as jnp
from jax import lax
from jax.experimental import pallas as pl
from jax.experimental.pallas import tpu as pltpu
from jax.experimental.pallas import tpu_sc as plsc

K_EMB = 8192
D_EMB = 256
M_BLK = 256
COMMIT_COST = 0.25


K_HALF = K_EMB // 2
N_ROWB = 32


def _vq_block(xn_ref, cn_ref, ct_ref, x_ref, idx_ref, enc_ref, loss_ref,
              acc_ref, run_min, run_idxf):
    i = pl.program_id(0)
    j = pl.program_id(1)

    @pl.when(j <= 1)
    def _compute():
        x = x_ref[...]
        ctj = ct_ref[:, pl.ds(j * K_HALF, K_HALF)]
        cnj = cn_ref[:, pl.ds(j * K_HALF, K_HALF)]
        mm = jnp.dot(x, ctj, preferred_element_type=jnp.float32)
        scores = (xn_ref[...] + cnj) - 2.0 * mm
        tmin = jnp.min(scores, axis=1)[:, None]
        kio = lax.broadcasted_iota(jnp.int32, (M_BLK, K_HALF), 1).astype(
            jnp.float32) + (j * K_HALF).astype(jnp.float32)
        cand = jnp.where(scores == tmin, kio, jnp.float32(jnp.inf))
        tidxf = jnp.min(cand, axis=1)[:, None]

        @pl.when(j == 0)
        def _first():
            run_min[...] = tmin
            run_idxf[...] = tidxf

        @pl.when(j == 1)
        def _rest():
            upd = tmin < run_min[...]
            run_idxf[...] = jnp.where(upd, tidxf, run_idxf[...])
            run_min[...] = jnp.minimum(run_min[...], tmin)

    @pl.when(j >= 2)
    def _publish_enc():
        kio = lax.broadcasted_iota(jnp.int32, (M_BLK, K_HALF), 1).astype(
            jnp.float32) + ((j - 2) * K_HALF).astype(jnp.float32)
        enc_ref[...] = (kio == run_idxf[...]).astype(jnp.float32)

    @pl.when(j == 2)
    def _publish_scalars():
        idx_ref[...] = run_idxf[...].astype(jnp.int32)

        @pl.when(i == 0)
        def _init():
            acc_ref[...] = jnp.zeros((M_BLK, 1), jnp.float32)

        acc_ref[...] += run_min[...]

    @pl.when((i == N_ROWB - 1) & (j == 3))
    def _fin():
        n_elems = jnp.float32(N_ROWB * M_BLK * D_EMB)
        loss_ref[0, 0] = COMMIT_COST * jnp.sum(acc_ref[...]) / n_elems


def _tc_vq(xn, cn, ct, flat):
    m = flat.shape[0]
    grid = (N_ROWB, 4)
    return pl.pallas_call(
        _vq_block,
        grid=grid,
        in_specs=[
            pl.BlockSpec((M_BLK, 1), lambda i, j: (i, 0)),
            pl.BlockSpec((1, K_EMB), lambda i, j: (0, 0)),
            pl.BlockSpec((D_EMB, K_EMB), lambda i, j: (0, 0)),
            pl.BlockSpec((M_BLK, D_EMB), lambda i, j: (i, 0)),
        ],
        out_specs=[
            pl.BlockSpec((M_BLK, 1), lambda i, j: (i, 0)),
            pl.BlockSpec((M_BLK, K_HALF), lambda i, j: (i, jnp.maximum(j - 2, 0))),
            pl.BlockSpec(memory_space=pltpu.SMEM),
        ],
        out_shape=[
            jax.ShapeDtypeStruct((m, 1), jnp.int32),
            jax.ShapeDtypeStruct((m, K_EMB), jnp.float32),
            jax.ShapeDtypeStruct((1, 1), jnp.float32),
        ],
        scratch_shapes=[
            pltpu.VMEM((M_BLK, 1), jnp.float32),
            pltpu.VMEM((M_BLK, 1), jnp.float32),
            pltpu.VMEM((M_BLK, 1), jnp.float32),
        ],
    )(xn, cn, ct, flat)


def _sc_gather(table, idx):
    info = plsc.get_sparse_core_info()
    nc, ns = info.num_cores, info.num_subcores
    nw = nc * ns
    b = idx.shape[0]
    d = table.shape[1]
    bpw = b // nw
    mesh = plsc.VectorSubcoreMesh(core_axis_name="c", subcore_axis_name="s")

    @functools.partial(
        pl.kernel,
        mesh=mesh,
        out_type=jax.ShapeDtypeStruct((b, d), jnp.float32),
        scratch_types=[
            pltpu.VMEM((bpw,), jnp.int32),
            pltpu.VMEM((bpw, d), jnp.float32),
            pltpu.SemaphoreType.DMA,
        ],
    )
    def gk(table_hbm, idx_hbm, out_hbm, idx_v, rows_v, sem):
        wid = lax.axis_index("s") * nc + lax.axis_index("c")
        base = wid * bpw
        pltpu.sync_copy(idx_hbm.at[pl.ds(base, bpw)], idx_v)
        pltpu.async_copy(table_hbm.at[idx_v], rows_v, sem).wait()
        pltpu.sync_copy(rows_v, out_hbm.at[pl.ds(base, bpw)])

    return gk(table, idx)


def kernel(inputs, codebook):
    s, n, d = inputs.shape
    flat = inputs.reshape(-1, d)
    xn = jnp.sum(flat ** 2, axis=1, keepdims=True)
    cn = jnp.sum(codebook ** 2, axis=1)[None, :]
    ct = codebook.T
    idx2, enc, loss2 = _tc_vq(xn, cn, ct, flat)
    idx = idx2.reshape(-1)
    quantized_st = _sc_gather(codebook, idx).reshape(s, n, d)
    loss = loss2[0, 0]
    return (loss, quantized_st, enc.reshape(s, n, K_EMB), idx2)


# speedup vs baseline: 1.4265x; 1.4265x over previous
import functools

import jax
import jax.numpy as jnp
from jax import lax
from jax.experimental import pallas as pl
from jax.experimental.pallas import tpu as pltpu
from jax.experimental.pallas import tpu_sc as plsc

K_EMB = 8192
D_EMB = 256
M_BLK = 256
COMMIT_COST = 0.25


def _vq_block(xn_ref, cn_ref, ct_ref, x_ref, idx_ref, enc_ref, loss_ref, acc_ref):
    i = pl.program_id(0)
    x = x_ref[...]
    ct = ct_ref[...]
    mm = jnp.dot(x, ct, preferred_element_type=jnp.float32)
    scores = (xn_ref[...] + cn_ref[...]) - 2.0 * mm
    minv = jnp.min(scores, axis=1)
    kiota_f = lax.broadcasted_iota(jnp.int32, (M_BLK, K_EMB), 1).astype(jnp.float32)
    cand = jnp.where(scores == minv[:, None], kiota_f, jnp.float32(jnp.inf))
    idxf = jnp.min(cand, axis=1)
    idx = idxf.astype(jnp.int32)
    idx_ref[...] = idx[:, None]
    enc_ref[...] = (kiota_f == idxf[:, None]).astype(jnp.float32)

    @pl.when(i == 0)
    def _init():
        acc_ref[...] = jnp.zeros((M_BLK, 1), jnp.float32)

    acc_ref[...] += minv[:, None]

    @pl.when(i == pl.num_programs(0) - 1)
    def _fin():
        n_elems = jnp.float32(pl.num_programs(0) * M_BLK * D_EMB)
        loss_ref[0, 0] = COMMIT_COST * jnp.sum(acc_ref[...]) / n_elems


def _tc_vq(xn, cn, ct, flat):
    m = flat.shape[0]
    grid = (m // M_BLK,)
    return pl.pallas_call(
        _vq_block,
        grid=grid,
        in_specs=[
            pl.BlockSpec((M_BLK, 1), lambda i: (i, 0)),
            pl.BlockSpec((1, K_EMB), lambda i: (0, 0)),
            pl.BlockSpec((D_EMB, K_EMB), lambda i: (0, 0)),
            pl.BlockSpec((M_BLK, D_EMB), lambda i: (i, 0)),
        ],
        out_specs=[
            pl.BlockSpec((M_BLK, 1), lambda i: (i, 0)),
            pl.BlockSpec((M_BLK, K_EMB), lambda i: (i, 0)),
            pl.BlockSpec(memory_space=pltpu.SMEM),
        ],
        out_shape=[
            jax.ShapeDtypeStruct((m, 1), jnp.int32),
            jax.ShapeDtypeStruct((m, K_EMB), jnp.float32),
            jax.ShapeDtypeStruct((1, 1), jnp.float32),
        ],
        scratch_shapes=[pltpu.VMEM((M_BLK, 1), jnp.float32)],
    )(xn, cn, ct, flat)


def _sc_gather(table, idx):
    info = plsc.get_sparse_core_info()
    nc, ns = info.num_cores, info.num_subcores
    nw = nc * ns
    b = idx.shape[0]
    d = table.shape[1]
    bpw = b // nw
    mesh = plsc.VectorSubcoreMesh(core_axis_name="c", subcore_axis_name="s")

    @functools.partial(
        pl.kernel,
        mesh=mesh,
        out_type=jax.ShapeDtypeStruct((b, d), jnp.float32),
        scratch_types=[
            pltpu.VMEM((bpw,), jnp.int32),
            pltpu.VMEM((bpw, d), jnp.float32),
            pltpu.SemaphoreType.DMA,
        ],
    )
    def gk(table_hbm, idx_hbm, out_hbm, idx_v, rows_v, sem):
        wid = lax.axis_index("s") * nc + lax.axis_index("c")
        base = wid * bpw
        pltpu.sync_copy(idx_hbm.at[pl.ds(base, bpw)], idx_v)
        pltpu.async_copy(table_hbm.at[idx_v], rows_v, sem).wait()
        pltpu.sync_copy(rows_v, out_hbm.at[pl.ds(base, bpw)])

    return gk(table, idx)


def kernel(inputs, codebook):
    s, n, d = inputs.shape
    flat = inputs.reshape(-1, d)
    xn = jnp.sum(flat ** 2, axis=1, keepdims=True)
    cn = jnp.sum(codebook ** 2, axis=1)[None, :]
    ct = codebook.T
    idx2, enc, loss2 = _tc_vq(xn, cn, ct, flat)
    idx = idx2.reshape(-1)
    quantized_st = _sc_gather(codebook, idx).reshape(s, n, d)
    loss = loss2[0, 0]
    return (loss, quantized_st, enc.reshape(s, n, K_EMB), idx2)
